# baseline (speedup 1.0000x reference)
import jax
import jax.numpy as jnp
from jax.experimental import pallas as pl
from jax.experimental.pallas import tpu as pltpu
from jax.experimental.pallas import tpu_sc as plsc

DIM = 32
C = 8192
BN = 128
ROWS = 16 * 1024
NB = ROWS // BN
GATHER_WINDOW = 256
PAD = 128


def _sim_body(x_ref, e_ref, dist_ref, ind_ref):
    x = x_ref[...]
    n = jnp.sqrt(jnp.sum(x * x, axis=-1, keepdims=True))
    xn = x / jnp.clip(n, 1e-12, None)
    e = e_ref[...]
    dist = jax.lax.dot_general(
        xn, e, (((1,), (1,)), ((), ())),
        preferred_element_type=jnp.float32,
    )
    dist_ref[...] = dist
    ind_ref[0, 0, :] = jnp.argmax(dist, axis=-1).astype(jnp.int32)


def _sc_gather(embed_pad, ind_flat):
    mesh = plsc.VectorSubcoreMesh(core_axis_name="core",
                                  subcore_axis_name="subcore")

    @pl.kernel(out_type=jax.ShapeDtypeStruct((ROWS, PAD), jnp.float32),
               mesh=mesh)
    def gather_kernel(e_hbm, i_hbm, o_hbm):
        def body(i_vmem, o_vmem):
            pltpu.sync_copy(e_hbm.at[i_vmem.at[0]], o_vmem)

        pltpu.emit_pipeline(
            body,
            grid=(ROWS // GATHER_WINDOW,),
            in_specs=[pl.BlockSpec((1, GATHER_WINDOW),
                                   index_map=lambda i: (0, i))],
            out_specs=[pl.BlockSpec((GATHER_WINDOW, PAD),
                                    index_map=lambda i: (i, 0))],
            core_axis_name=("core", "subcore"),
            dimension_semantics=(pltpu.PARALLEL,),
        )(i_hbm, o_hbm)

    return gather_kernel(embed_pad, ind_flat)


@jax.jit
def kernel(x, embed):
    b, npts, d = x.shape
    xf = x.reshape(b * npts, d)
    e2 = embed[0]
    dist, ind3 = pl.pallas_call(
        _sim_body,
        grid=(NB,),
        in_specs=[
            pl.BlockSpec((BN, DIM), lambda i: (i, 0)),
            pl.BlockSpec((C, DIM), lambda i: (0, 0)),
        ],
        out_specs=[
            pl.BlockSpec((BN, C), lambda i: (i, 0)),
            pl.BlockSpec((1, 1, BN), lambda i: (i, 0, 0)),
        ],
        out_shape=[
            jax.ShapeDtypeStruct((ROWS, C), jnp.float32),
            jax.ShapeDtypeStruct((NB, 1, BN), jnp.int32),
        ],
    )(xf, e2)
    e_pad = jnp.pad(e2, ((0, 0), (0, PAD - DIM)))
    q = _sc_gather(e_pad, ind3.reshape(1, ROWS))[:, :DIM]
    return (q.reshape(b, npts, d),
            ind3.reshape(b, npts),
            dist.reshape(b, npts, C))


# speedup vs baseline: 1.1167x; 1.1167x over previous
import jax
import jax.numpy as jnp
from jax.experimental import pallas as pl
from jax.experimental.pallas import tpu as pltpu
from jax.experimental.pallas import tpu_sc as plsc

DIM = 32
C = 8192
BN = 256
ROWS = 16 * 1024
NB = ROWS // BN
GATHER_WINDOW = 256
PAD = 128


def _sim_body(x_ref, e_ref, dist_ref, ind_ref):
    x = x_ref[...]
    n = jnp.sqrt(jnp.sum(x * x, axis=-1, keepdims=True))
    xn = x / jnp.clip(n, 1e-12, None)
    e = e_ref[...]
    dist = jax.lax.dot_general(
        xn, e, (((1,), (1,)), ((), ())),
        preferred_element_type=jnp.float32,
    )
    dist_ref[...] = dist
    ind_ref[0, 0, :] = jnp.argmax(dist, axis=-1).astype(jnp.int32)


def _sc_gather(embed_pad, ind_flat):
    mesh = plsc.VectorSubcoreMesh(core_axis_name="core",
                                  subcore_axis_name="subcore")

    @pl.kernel(out_type=jax.ShapeDtypeStruct((ROWS, PAD), jnp.float32),
               mesh=mesh)
    def gather_kernel(e_hbm, i_hbm, o_hbm):
        def body(i_vmem, o_vmem):
            pltpu.sync_copy(e_hbm.at[i_vmem.at[0]], o_vmem)

        pltpu.emit_pipeline(
            body,
            grid=(ROWS // GATHER_WINDOW,),
            in_specs=[pl.BlockSpec((1, GATHER_WINDOW),
                                   index_map=lambda i: (0, i))],
            out_specs=[pl.BlockSpec((GATHER_WINDOW, PAD),
                                    index_map=lambda i: (i, 0))],
            core_axis_name=("core", "subcore"),
            dimension_semantics=(pltpu.PARALLEL,),
        )(i_hbm, o_hbm)

    return gather_kernel(embed_pad, ind_flat)


@jax.jit
def kernel(x, embed):
    b, npts, d = x.shape
    xf = x.reshape(b * npts, d)
    e2 = embed[0]
    dist, ind3 = pl.pallas_call(
        _sim_body,
        grid=(NB,),
        in_specs=[
            pl.BlockSpec((BN, DIM), lambda i: (i, 0)),
            pl.BlockSpec((C, DIM), lambda i: (0, 0)),
        ],
        out_specs=[
            pl.BlockSpec((BN, C), lambda i: (i, 0)),
            pl.BlockSpec((1, 1, BN), lambda i: (i, 0, 0)),
        ],
        out_shape=[
            jax.ShapeDtypeStruct((ROWS, C), jnp.float32),
            jax.ShapeDtypeStruct((NB, 1, BN), jnp.int32),
        ],
    )(xf, e2)
    e_pad = jnp.pad(e2, ((0, 0), (0, PAD - DIM)))
    q = _sc_gather(e_pad, ind3.reshape(1, ROWS))[:, :DIM]
    return (q.reshape(b, npts, d),
            ind3.reshape(b, npts),
            dist.reshape(b, npts, C))


# speedup vs baseline: 1.1168x; 1.0001x over previous
import jax
import jax.numpy as jnp
from jax.experimental import pallas as pl
from jax.experimental.pallas import tpu as pltpu
from jax.experimental.pallas import tpu_sc as plsc

DIM = 32
C = 8192
BN = 256
ROWS = 16 * 1024
NB = ROWS // BN
GATHER_WINDOW = 256
PAD = 128


def _sim_body(x_ref, e_ref, dist_ref, ind_ref):
    x = x_ref[...]
    n = jnp.sqrt(jnp.sum(x * x, axis=-1, keepdims=True))
    xn = x / jnp.clip(n, 1e-12, None)
    e = e_ref[...]
    dist = jax.lax.dot_general(
        xn, e, (((1,), (1,)), ((), ())),
        preferred_element_type=jnp.float32,
    )
    ind_ref[0, 0, :] = jnp.argmax(dist, axis=-1).astype(jnp.int32)
    dist_ref[...] = dist


def _sc_gather(embed_pad, ind_flat):
    mesh = plsc.VectorSubcoreMesh(core_axis_name="core",
                                  subcore_axis_name="subcore")

    @pl.kernel(out_type=jax.ShapeDtypeStruct((ROWS, PAD), jnp.float32),
               mesh=mesh)
    def gather_kernel(e_hbm, i_hbm, o_hbm):
        def body(i_vmem, o_vmem):
            pltpu.sync_copy(e_hbm.at[i_vmem.at[0]], o_vmem)

        pltpu.emit_pipeline(
            body,
            grid=(ROWS // GATHER_WINDOW,),
            in_specs=[pl.BlockSpec((1, GATHER_WINDOW),
                                   index_map=lambda i: (0, i))],
            out_specs=[pl.BlockSpec((GATHER_WINDOW, PAD),
                                    index_map=lambda i: (i, 0))],
            core_axis_name=("core", "subcore"),
            dimension_semantics=(pltpu.PARALLEL,),
        )(i_hbm, o_hbm)

    return gather_kernel(embed_pad, ind_flat)


@jax.jit
def kernel(x, embed):
    b, npts, d = x.shape
    xf = x.reshape(b * npts, d)
    e2 = embed[0]
    dist, ind3 = pl.pallas_call(
        _sim_body,
        grid=(NB,),
        in_specs=[
            pl.BlockSpec((BN, DIM), lambda i: (i, 0)),
            pl.BlockSpec((C, DIM), lambda i: (0, 0)),
        ],
        out_specs=[
            pl.BlockSpec((BN, C), lambda i: (i, 0)),
            pl.BlockSpec((1, 1, BN), lambda i: (i, 0, 0)),
        ],
        out_shape=[
            jax.ShapeDtypeStruct((ROWS, C), jnp.float32),
            jax.ShapeDtypeStruct((NB, 1, BN), jnp.int32),
        ],
    )(xf, e2)
    e_pad = jnp.pad(e2, ((0, 0), (0, PAD - DIM)))
    q = _sc_gather(e_pad, ind3.reshape(1, ROWS))[:, :DIM]
    return (q.reshape(b, npts, d),
            ind3.reshape(b, npts),
            dist.reshape(b, npts, C))
